# Initial kernel scaffold; baseline (speedup 1.0000x reference)
import jax
import jax.numpy as jnp
from jax.experimental import pallas as pl


def kernel(source_user_emb, source_item_emb, target_user_emb, target_item_emb, head_s, tail_s, head_t, tail_t):
    raise NotImplementedError("write your pallas kernel here")


# speedup vs baseline: 13.8565x; 13.8565x over previous
import functools

import jax
import jax.numpy as jnp
from jax import lax
from jax.experimental import pallas as pl
from jax.experimental.pallas import tpu as pltpu
from jax.experimental.pallas import tpu_sc as plsc

N_USER = 5000
N_NODE = 10000
NSUB = 16
NS = 10240
ROWS_W = NS // NSUB
E = 320000
E_W = E // NSUB
KB = 80
NBLK = E_W // KB
EMB = 128
F = 4
C = 32
G = 2

_f32 = jnp.float32
_i32 = jnp.int32


def _mesh():
    return plsc.VectorSubcoreMesh(core_axis_name="c", subcore_axis_name="s")


@functools.partial(
    pl.kernel, mesh=_mesh(),
    out_type=jax.ShapeDtypeStruct((G * NS,), _f32),
    scratch_types=[
        pltpu.VMEM((1, KB), _i32),
        pltpu.VMEM((KB,), _f32),
        pltpu.VMEM_SHARED((NS,), _f32),
    ],
)
def _sc_hist(head_l, z1, out, hidx_v, ones_v, acc_sh):
    g = lax.axis_index("c")
    s = lax.axis_index("s")
    r0 = s * ROWS_W
    for k in range(KB // 16):
        ones_v[pl.ds(16 * k, 16)] = jnp.ones((16,), _f32)
    pltpu.sync_copy(z1.at[pl.ds(0, ROWS_W)], acc_sh.at[pl.ds(r0, ROWS_W)])
    plsc.subcore_barrier()
    ebase = g * E + s * E_W

    def body(j, carry):
        off = ebase + j * KB
        pltpu.sync_copy(head_l.at[pl.ds(off, KB)], hidx_v.at[0])
        pltpu.sync_copy(ones_v, acc_sh.at[hidx_v.at[0]], add=True)
        return carry

    lax.fori_loop(0, NBLK, body, 0)
    plsc.subcore_barrier()
    pltpu.sync_copy(acc_sh.at[pl.ds(r0, ROWS_W)],
                    out.at[pl.ds(g * NS + r0, ROWS_W)])


@functools.partial(
    pl.kernel, mesh=_mesh(),
    out_type=jax.ShapeDtypeStruct((G * NS, EMB), _f32),
    scratch_types=[
        pltpu.VMEM((KB,), _i32),
        pltpu.VMEM((1, KB), _i32),
        pltpu.VMEM((KB, EMB), _f32),
        pltpu.VMEM_SHARED((NS, EMB), _f32),
        pltpu.SemaphoreType.DMA,
    ],
)
def _sc_gs0(tail_g, head_l, table, z128, out, tidx_v, hidx_v, rows_v, acc_sh,
            sem):
    g = lax.axis_index("c")
    s = lax.axis_index("s")
    r0 = s * ROWS_W
    pltpu.sync_copy(z128.at[pl.ds(0, ROWS_W)], acc_sh.at[pl.ds(r0, ROWS_W)])
    plsc.subcore_barrier()
    ebase = g * E + s * E_W

    def body(j, carry):
        off = ebase + j * KB
        pltpu.sync_copy(tail_g.at[pl.ds(off, KB)], tidx_v)
        pltpu.sync_copy(head_l.at[pl.ds(off, KB)], hidx_v.at[0])
        pltpu.async_copy(table.at[tidx_v], rows_v, sem).wait()
        pltpu.sync_copy(rows_v, acc_sh.at[hidx_v.at[0]], add=True)
        return carry

    lax.fori_loop(0, NBLK, body, 0)
    plsc.subcore_barrier()
    pltpu.sync_copy(acc_sh.at[pl.ds(r0, ROWS_W)],
                    out.at[pl.ds(g * NS + r0, ROWS_W)])


@functools.partial(
    pl.kernel, mesh=_mesh(),
    out_type=[jax.ShapeDtypeStruct((G * E, EMB), _f32),
              jax.ShapeDtypeStruct((G * E, EMB), _f32)],
    scratch_types=[
        pltpu.VMEM((KB,), _i32),
        pltpu.VMEM((KB,), _i32),
        pltpu.VMEM((KB, EMB), _f32),
        pltpu.VMEM((KB, EMB), _f32),
        pltpu.SemaphoreType.DMA,
    ],
)
def _sc_gather2(head_g, tail_g, htab, ttab, outh, outt, hidx_v, tidx_v,
                hrows_v, trows_v, sem):
    g = lax.axis_index("c")
    s = lax.axis_index("s")
    ebase = g * E + s * E_W

    def body(j, carry):
        off = ebase + j * KB
        pltpu.sync_copy(head_g.at[pl.ds(off, KB)], hidx_v)
        pltpu.sync_copy(tail_g.at[pl.ds(off, KB)], tidx_v)
        pltpu.async_copy(htab.at[hidx_v], hrows_v, sem).wait()
        pltpu.sync_copy(hrows_v, outh.at[pl.ds(off, KB)])
        pltpu.async_copy(ttab.at[tidx_v], trows_v, sem).wait()
        pltpu.sync_copy(trows_v, outt.at[pl.ds(off, KB)])
        return carry

    lax.fori_loop(0, NBLK, body, 0)


_KD = 128
_NBLK_D = E_W * F // _KD
_RW4 = ROWS_W * F


@functools.partial(
    pl.kernel, mesh=_mesh(),
    out_type=jax.ShapeDtypeStruct((G * NS * F,), _f32),
    scratch_types=[
        pltpu.VMEM((1, _KD), _i32),
        pltpu.VMEM((_KD,), _f32),
        pltpu.VMEM_SHARED((NS * F,), _f32),
    ],
)
def _sc_deg(norm_af, head4_l, z1, out, hidx_v, w_v, acc_sh):
    g = lax.axis_index("c")
    s = lax.axis_index("s")
    r0 = s * _RW4
    pltpu.sync_copy(z1.at[pl.ds(0, _RW4)], acc_sh.at[pl.ds(r0, _RW4)])
    plsc.subcore_barrier()
    ebase = (g * E + s * E_W) * F

    def body(j, carry):
        off = ebase + j * _KD
        pltpu.sync_copy(norm_af.at[pl.ds(off, _KD)], w_v)
        pltpu.sync_copy(head4_l.at[pl.ds(off, _KD)], hidx_v.at[0])
        pltpu.sync_copy(w_v, acc_sh.at[hidx_v.at[0]], add=True)
        return carry

    lax.fori_loop(0, _NBLK_D, body, 0)
    plsc.subcore_barrier()
    pltpu.sync_copy(acc_sh.at[pl.ds(r0, _RW4)],
                    out.at[pl.ds(g * NS * F + r0, _RW4)])


@functools.partial(
    pl.kernel, mesh=_mesh(),
    out_type=jax.ShapeDtypeStruct((G * NS, EMB), _f32),
    scratch_types=[
        pltpu.VMEM((KB,), _i32),
        pltpu.VMEM((1, KB), _i32),
        pltpu.VMEM((KB * F,), _f32),
        pltpu.VMEM((KB, EMB), _f32),
        pltpu.VMEM_SHARED((NS, EMB), _f32),
        pltpu.SemaphoreType.DMA,
    ],
)
def _sc_gs1(tail_g, head_l, table, norm_af, z128, out, tidx_v, hidx_v, w_v,
            rows_v, acc_sh, sem):
    g = lax.axis_index("c")
    s = lax.axis_index("s")
    r0 = s * ROWS_W
    pltpu.sync_copy(z128.at[pl.ds(0, ROWS_W)], acc_sh.at[pl.ds(r0, ROWS_W)])
    plsc.subcore_barrier()
    ebase = g * E + s * E_W

    def body(j, carry):
        off = ebase + j * KB
        pltpu.sync_copy(tail_g.at[pl.ds(off, KB)], tidx_v)
        pltpu.sync_copy(head_l.at[pl.ds(off, KB)], hidx_v.at[0])
        pltpu.sync_copy(norm_af.at[pl.ds(off * F, KB * F)], w_v)
        pltpu.async_copy(table.at[tidx_v], rows_v, sem).wait()
        for t in range(KB // 4):
            wq = w_v[pl.ds(16 * t, 16)]
            for r in range(4):
                e = 4 * t + r
                for i in range(F):
                    wb = jnp.broadcast_to(wq[4 * r + i], (16,))
                    s0 = pl.ds(i * C, 16)
                    s1 = pl.ds(i * C + 16, 16)
                    rows_v[e, s0] = rows_v[e, s0] * wb
                    rows_v[e, s1] = rows_v[e, s1] * wb
        pltpu.sync_copy(rows_v, acc_sh.at[hidx_v.at[0]], add=True)
        return carry

    lax.fori_loop(0, NBLK, body, 0)
    plsc.subcore_barrier()
    pltpu.sync_copy(acc_sh.at[pl.ds(r0, ROWS_W)],
                    out.at[pl.ds(g * NS + r0, ROWS_W)])


_BR = 256
_BE = 1024


def _tc1_body(cnt_ref, ego_ref, md_ref, mu_ref, egos_ref, ttab_ref):
    cnt = cnt_ref[...]
    ego = ego_ref[...]
    dinv0 = lax.rsqrt(jnp.maximum(0.25 * cnt, 1e-8))
    egos_ref[...] = (0.25 * dinv0) * ego
    bs = jnp.dot(ego * ego, md_ref[...], preferred_element_type=_f32)
    ninv = 1.0 / jnp.maximum(jnp.sqrt(bs), 1e-12)
    nexp = jnp.dot(ninv, mu_ref[...], preferred_element_type=_f32)
    ttab_ref[...] = jnp.tanh(ego * nexp)


def _tc2_body(cnt_ref, f0_ref, md_ref, mu_ref, hn_ref):
    cnt = cnt_ref[...]
    dinv0 = lax.rsqrt(jnp.maximum(0.25 * cnt, 1e-8))
    f0 = f0_ref[...] * dinv0
    bs = jnp.dot(f0 * f0, md_ref[...], preferred_element_type=_f32)
    ninv = 1.0 / jnp.maximum(jnp.sqrt(bs), 1e-12)
    hn_ref[...] = f0 * jnp.dot(ninv, mu_ref[...], preferred_element_type=_f32)


def _tc3_body(h_ref, t_ref, md_ref, na_ref):
    a = 1.0 + jnp.dot(h_ref[...] * t_ref[...], md_ref[...],
                      preferred_element_type=_f32)
    m = jnp.max(a, axis=1, keepdims=True)
    ea = jnp.exp(a - m)
    na_ref[...] = ea / jnp.sum(ea, axis=1, keepdims=True)


def _tc4_body(deg_ref, ego_ref, mu_ref, egot_ref):
    dinv1 = lax.rsqrt(jnp.maximum(deg_ref[...], 1e-8))
    egot_ref[...] = ego_ref[...] * jnp.dot(dinv1, mu_ref[...],
                                           preferred_element_type=_f32)


def _tc5_body(deg_ref, ego_ref, f1_ref, mu_ref, out_ref):
    dinv1 = lax.rsqrt(jnp.maximum(deg_ref[...], 1e-8))
    dexp = jnp.dot(dinv1, mu_ref[...], preferred_element_type=_f32)
    out_ref[...] = 0.5 * (ego_ref[...] + f1_ref[...] * dexp)


def _row_spec(w):
    return pl.BlockSpec((_BR, w), lambda i: (i, 0))


def _full_spec(shape):
    return pl.BlockSpec(shape, lambda i: (0, 0))


def _tc_call(body, in_specs, out_specs, out_shape, grid):
    return pl.pallas_call(body, grid=(grid,), in_specs=in_specs,
                          out_specs=out_specs, out_shape=out_shape)


def kernel(source_user_emb, source_item_emb, target_user_emb, target_item_emb,
           head_s, tail_s, head_t, tail_t):
    ego = jnp.stack([jnp.concatenate([source_user_emb, source_item_emb]),
                     jnp.concatenate([target_user_emb, target_item_emb])])
    ego = jnp.concatenate(
        [ego, jnp.zeros((G, NS - N_NODE, EMB), _f32)], axis=1)
    ego_flat = ego.reshape(G * NS, EMB)
    head = jnp.concatenate([head_s, head_t]).astype(_i32)
    tail = jnp.concatenate([tail_s, tail_t]).astype(_i32)
    goff = jnp.repeat(jnp.arange(G, dtype=_i32) * NS, E)
    head_g = head + goff
    tail_g = tail + goff
    z1 = jnp.zeros((NS,), _f32)
    z128 = jnp.zeros((NS, EMB), _f32)
    dmask = (jnp.arange(EMB)[:, None] // C == jnp.arange(F)[None, :])
    md = dmask.astype(_f32)
    mu = md.T

    ng = G * NS // _BR
    eg = G * E // _BE

    cnt = _sc_hist(head, z1)
    cnt_c = cnt.reshape(G * NS, 1)
    egos, ttab = _tc_call(
        _tc1_body,
        [_row_spec(1), _row_spec(EMB), _full_spec((EMB, F)),
         _full_spec((F, EMB))],
        [_row_spec(EMB), _row_spec(EMB)],
        [jax.ShapeDtypeStruct((G * NS, EMB), _f32)] * 2,
        ng)(cnt_c, ego_flat, md, mu)
    femb0s = _sc_gs0(tail_g, head, egos, z128)
    hnorm = _tc_call(
        _tc2_body,
        [_row_spec(1), _row_spec(EMB), _full_spec((EMB, F)),
         _full_spec((F, EMB))],
        _row_spec(EMB),
        jax.ShapeDtypeStruct((G * NS, EMB), _f32),
        ng)(cnt_c, femb0s, md, mu)
    hg, tg = _sc_gather2(head_g, tail_g, hnorm, ttab)
    norm_a = _tc_call(
        _tc3_body,
        [pl.BlockSpec((_BE, EMB), lambda i: (i, 0)),
         pl.BlockSpec((_BE, EMB), lambda i: (i, 0)), _full_spec((EMB, F))],
        pl.BlockSpec((_BE, F), lambda i: (i, 0)),
        jax.ShapeDtypeStruct((G * E, F), _f32),
        eg)(hg, tg, md)
    head4 = (head[:, None] * F + jnp.arange(F, dtype=_i32)).reshape(-1)
    norm_af = norm_a.reshape(-1)
    deg1_c = _sc_deg(norm_af, head4, z1).reshape(G * NS, F)
    egot = _tc_call(
        _tc4_body,
        [_row_spec(F), _row_spec(EMB), _full_spec((F, EMB))],
        _row_spec(EMB),
        jax.ShapeDtypeStruct((G * NS, EMB), _f32),
        ng)(deg1_c, ego_flat, mu)
    femb1s = _sc_gs1(tail_g, head, egot, norm_af, z128)
    outf = _tc_call(
        _tc5_body,
        [_row_spec(F), _row_spec(EMB), _row_spec(EMB), _full_spec((F, EMB))],
        _row_spec(EMB),
        jax.ShapeDtypeStruct((G * NS, EMB), _f32),
        ng)(deg1_c, ego_flat, femb1s, mu)
    out = outf.reshape(G, NS, EMB)
    return (out[0, :N_USER], out[0, N_USER:N_NODE],
            out[1, :N_USER], out[1, N_USER:N_NODE])
